# Initial kernel scaffold; baseline (speedup 1.0000x reference)
import jax
import jax.numpy as jnp
from jax.experimental import pallas as pl


def kernel(x):
    raise NotImplementedError("write your pallas kernel here")


# speedup vs baseline: 1.8189x; 1.8189x over previous
import jax
import jax.numpy as jnp
import numpy as np
from jax import lax
from jax.experimental import pallas as pl
from jax.experimental.pallas import tpu as pltpu
from jax.experimental.pallas import tpu_sc as plsc

B, S, D = 4, 4096, 1024
ROWS = B * S
NUM_WORKERS = 32
SLAB = ROWS // NUM_WORKERS
SLABS_PER_BATCH = S // SLAB
CHUNK = 32
NCHUNK = SLAB // CHUNK
CGRP = D // 16

_T_CHANGE = np.uint32(838860)
_T_MINUS = np.uint32(419430)


def _threefry_bits(p):
    ks0 = np.uint32(0)
    ks1 = np.uint32(42)
    ks2 = np.uint32(0 ^ 42 ^ 0x1BD11BDA)

    def rotl(x, r):
        return lax.shift_left(x, np.uint32(r)) | lax.shift_right_logical(
            x, np.uint32(32 - r))

    x0 = jnp.zeros_like(p)
    x1 = p + ks1
    rot_a = (13, 15, 26, 6)
    rot_b = (17, 29, 16, 24)
    sched = ((ks1, ks2, 1), (ks2, ks0, 2), (ks0, ks1, 3), (ks1, ks2, 4),
             (ks2, ks0, 5))
    for g, (a0, a1, c) in enumerate(sched):
        for r in (rot_a if g % 2 == 0 else rot_b):
            x0 = x0 + x1
            x1 = x0 ^ rotl(x1, r)
        x0 = x0 + a0
        x1 = x1 + np.uint32(a1 + np.uint32(c))
    return x0 ^ x1


def _jitter_body(x_hbm, out_hbm, in_buf, out_buf):
    nc = 2
    wid = lax.axis_index("s") * nc + lax.axis_index("c")
    slab0 = wid * SLAB
    at_batch_lo = lax.rem(wid, SLABS_PER_BATCH) == 0
    at_batch_hi = lax.rem(wid, SLABS_PER_BATCH) == SLABS_PER_BATCH - 1
    lane = lax.iota(jnp.uint32, 16)

    def chunk_body(chunk, _):
        g0 = slab0 + chunk * CHUNK
        is_first = jnp.logical_and(at_batch_lo, chunk == 0)
        is_last = jnp.logical_and(at_batch_hi, chunk == NCHUNK - 1)
        interior = jnp.logical_and(jnp.logical_not(is_first),
                                   jnp.logical_not(is_last))

        @pl.when(is_first)
        def _():
            pltpu.sync_copy(x_hbm.at[pl.ds(g0, CHUNK + 1)],
                            in_buf.at[pl.ds(1, CHUNK + 1)])
            pltpu.sync_copy(x_hbm.at[pl.ds(g0, 1)], in_buf.at[pl.ds(0, 1)])

        @pl.when(is_last)
        def _():
            pltpu.sync_copy(x_hbm.at[pl.ds(g0 - 1, CHUNK + 1)],
                            in_buf.at[pl.ds(0, CHUNK + 1)])
            pltpu.sync_copy(x_hbm.at[pl.ds(g0 + CHUNK - 1, 1)],
                            in_buf.at[pl.ds(CHUNK + 1, 1)])

        @pl.when(interior)
        def _():
            pltpu.sync_copy(x_hbm.at[pl.ds(g0 - 1, CHUNK + 2)],
                            in_buf.at[pl.ds(0, CHUNK + 2)])

        p_chunk0 = lax.convert_element_type(g0 * D, jnp.uint32)

        def vec_body(i, _):
            s_local = lax.shift_right_logical(i, 6)
            c16 = lax.shift_left(lax.bitwise_and(i, 63), 4)
            p = (p_chunk0 + lax.convert_element_type(s_local * D + c16,
                                                     jnp.uint32)) + lane
            m = lax.shift_right_logical(_threefry_bits(p), np.uint32(9))
            v_dn = in_buf[s_local, pl.ds(c16, 16)]
            v_mid = in_buf[s_local + 1, pl.ds(c16, 16)]
            v_up = in_buf[s_local + 2, pl.ds(c16, 16)]
            moved = jnp.where(m <= _T_MINUS, v_dn, v_up)
            out_buf[s_local, pl.ds(c16, 16)] = jnp.where(
                m <= _T_CHANGE, moved, v_mid)
            return _

        lax.fori_loop(0, CHUNK * CGRP, vec_body, None, unroll=2)
        pltpu.sync_copy(out_buf, out_hbm.at[pl.ds(g0, CHUNK)])
        return _

    lax.fori_loop(0, NCHUNK, chunk_body, None)


@jax.jit
def kernel(x):
    x2d = x.reshape(ROWS, D)
    mesh = plsc.VectorSubcoreMesh(core_axis_name="c", subcore_axis_name="s")
    out = pl.kernel(
        _jitter_body,
        mesh=mesh,
        out_type=jax.ShapeDtypeStruct((ROWS, D), jnp.float32),
        compiler_params=pltpu.CompilerParams(use_tc_tiling_on_sc=False),
        scratch_types=[
            pltpu.VMEM((CHUNK + 2, D), jnp.float32),
            pltpu.VMEM((CHUNK, D), jnp.float32),
        ],
    )(x2d)
    return out.reshape(B, S, D)
